# Initial kernel scaffold; baseline (speedup 1.0000x reference)
import jax
import jax.numpy as jnp
from jax.experimental import pallas as pl


def kernel(hidden_preactivation_BH):
    raise NotImplementedError("write your pallas kernel here")


# speedup vs baseline: 17.4592x; 17.4592x over previous
import functools

import jax
import jax.numpy as jnp
from jax.experimental import pallas as pl
from jax.experimental.pallas import tpu as pltpu

_TOPK_PER_ROW = 64


def _select_body(total_k, chunk_rows, x_ref, out_ref, ku_ref):
    b, h = x_ref.shape
    n = b * h
    kk = jnp.int32(total_k)
    nch = b // chunk_rows

    def conv(ci, _):
        xc = x_ref[pl.ds(ci * chunk_rows, chunk_rows), :]
        i = jax.lax.bitcast_convert_type(xc, jnp.int32)
        k = i ^ ((i >> 31) & jnp.int32(0x7FFFFFFF))
        ku_ref[pl.ds(ci * chunk_rows, chunk_rows), :] = (
            jax.lax.bitcast_convert_type(k, jnp.uint32) ^ jnp.uint32(0x80000000)
        )
        return 0

    jax.lax.fori_loop(0, nch, conv, 0, unroll=True)

    def count_ge(c):
        def body(ci, acc):
            kc = ku_ref[pl.ds(ci * chunk_rows, chunk_rows), :]
            return acc + jnp.sum((kc >= c).astype(jnp.int32))

        return jax.lax.fori_loop(0, nch, body, jnp.int32(0), unroll=True)

    def step(it, carry):
        p, cnt = carry
        cand = p | (jnp.uint32(1) << (jnp.uint32(31) - it.astype(jnp.uint32)))
        c = count_ge(cand)
        take = c >= kk
        return (jnp.where(take, cand, p), jnp.where(take, c, cnt))

    t, c_ge = jax.lax.fori_loop(0, 32, step, (jnp.uint32(0), jnp.int32(n)))

    def count_eq(c):
        def body(ci, acc):
            kc = ku_ref[pl.ds(ci * chunk_rows, chunk_rows), :]
            return acc + jnp.sum((kc == c).astype(jnp.int32))

        return jax.lax.fori_loop(0, nch, body, jnp.int32(0), unroll=True)

    tie_count = count_eq(t)
    n_keep_ties = kk - (c_ge - tie_count)
    all_ties = n_keep_ties == tie_count

    @pl.when(all_ties)
    def _():
        def wr(ci, _):
            sl = pl.ds(ci * chunk_rows, chunk_rows)
            out_ref[sl, :] = jnp.where(ku_ref[sl, :] >= t, x_ref[sl, :], 0.0)
            return 0

        jax.lax.fori_loop(0, nch, wr, 0, unroll=True)

    @pl.when(jnp.logical_not(all_ties))
    def _():
        def chunk_idx(ci):
            return (
                (jax.lax.broadcasted_iota(jnp.int32, (chunk_rows, h), 0)
                 + ci * chunk_rows) * jnp.int32(h)
                + jax.lax.broadcasted_iota(jnp.int32, (chunk_rows, h), 1)
            )

        def istep(it, p2):
            cand = p2 | (jnp.int32(1) << (jnp.int32(31) - it))

            def body(ci, acc):
                kc = ku_ref[pl.ds(ci * chunk_rows, chunk_rows), :]
                m = (kc == t) & (chunk_idx(ci) < cand)
                return acc + jnp.sum(m.astype(jnp.int32))

            c = jax.lax.fori_loop(0, nch, body, jnp.int32(0))
            return jnp.where(c >= n_keep_ties, p2, cand)

        m = jax.lax.fori_loop(1, 32, istep, jnp.int32(0))

        def wr(ci, _):
            sl = pl.ds(ci * chunk_rows, chunk_rows)
            kc = ku_ref[sl, :]
            keep = (kc > t) | ((kc == t) & (chunk_idx(ci) <= m))
            out_ref[sl, :] = jnp.where(keep, x_ref[sl, :], 0.0)
            return 0

        jax.lax.fori_loop(0, nch, wr, 0, unroll=True)


def _batch_topk(x, total_k):
    b, h = x.shape
    return pl.pallas_call(
        functools.partial(_select_body, total_k, 16),
        out_shape=jax.ShapeDtypeStruct(x.shape, x.dtype),
        scratch_shapes=[pltpu.VMEM((b, h), jnp.uint32)],
    )(x)


@jax.jit
def kernel(hidden_preactivation_BH):
    b = hidden_preactivation_BH.shape[0]
    return _batch_topk(hidden_preactivation_BH, _TOPK_PER_ROW * b)
